# Initial kernel scaffold; baseline (speedup 1.0000x reference)
import jax
import jax.numpy as jnp
from jax.experimental import pallas as pl


def kernel(sample, epoch, epoch_table):
    raise NotImplementedError("write your pallas kernel here")


# speedup vs baseline: 1.2465x; 1.2465x over previous
import jax
import jax.numpy as jnp
from jax.experimental import pallas as pl
from jax.experimental.pallas import tpu as pltpu

_BLOCK = 2048


def _concat_kernel(epoch_ref, table_ref, sample_ref, out_ref):
    e = epoch_ref[0]
    row = table_ref[pl.ds(e, 1), :]
    nf = sample_ref.shape[1]
    out_ref[:, :nf] = sample_ref[...]
    out_ref[:, nf:] = jnp.broadcast_to(row, (out_ref.shape[0], row.shape[1]))


def kernel(sample, epoch, epoch_table):
    batch, nfeat = sample.shape
    nvocab, nemb = epoch_table.shape
    epoch_arr = jnp.asarray(epoch, jnp.int32).reshape((1,))
    nout = nfeat + nemb
    grid = (batch // _BLOCK,)
    return pl.pallas_call(
        _concat_kernel,
        grid=grid,
        in_specs=[
            pl.BlockSpec(memory_space=pltpu.SMEM),
            pl.BlockSpec((nvocab, nemb), lambda i: (0, 0)),
            pl.BlockSpec((_BLOCK, nfeat), lambda i: (i, 0)),
        ],
        out_specs=pl.BlockSpec((_BLOCK, nout), lambda i: (i, 0)),
        out_shape=jax.ShapeDtypeStruct((batch, nout), sample.dtype),
        compiler_params=pltpu.CompilerParams(
            dimension_semantics=("arbitrary",),
        ),
    )(epoch_arr, epoch_table, sample)


# speedup vs baseline: 1.3096x; 1.0507x over previous
import jax
import jax.numpy as jnp
from jax.experimental import pallas as pl
from jax.experimental.pallas import tpu as pltpu

_BLOCK = 4096


def _concat_kernel(epoch_ref, table_ref, sample_ref, out_ref):
    e = epoch_ref[0]
    row = table_ref[pl.ds(e, 1), :]
    nf = sample_ref.shape[1]
    out_ref[:, :nf] = sample_ref[...]
    out_ref[:, nf:] = jnp.broadcast_to(row, (out_ref.shape[0], row.shape[1]))


def kernel(sample, epoch, epoch_table):
    batch, nfeat = sample.shape
    nvocab, nemb = epoch_table.shape
    epoch_arr = jnp.asarray(epoch, jnp.int32).reshape((1,))
    nout = nfeat + nemb
    grid = (batch // _BLOCK,)
    return pl.pallas_call(
        _concat_kernel,
        grid=grid,
        in_specs=[
            pl.BlockSpec(memory_space=pltpu.SMEM),
            pl.BlockSpec((nvocab, nemb), lambda i: (0, 0)),
            pl.BlockSpec((_BLOCK, nfeat), lambda i: (i, 0)),
        ],
        out_specs=pl.BlockSpec((_BLOCK, nout), lambda i: (i, 0)),
        out_shape=jax.ShapeDtypeStruct((batch, nout), sample.dtype),
        compiler_params=pltpu.CompilerParams(
            dimension_semantics=("arbitrary",),
        ),
    )(epoch_arr, epoch_table, sample)


# speedup vs baseline: 1.3565x; 1.0358x over previous
import jax
import jax.numpy as jnp
from jax.experimental import pallas as pl
from jax.experimental.pallas import tpu as pltpu

_BLOCK = 8192


def _concat_kernel(epoch_ref, table_ref, sample_ref, out_ref):
    e = epoch_ref[0]
    row = table_ref[pl.ds(e, 1), :]
    nf = sample_ref.shape[1]
    out_ref[:, :nf] = sample_ref[...]
    out_ref[:, nf:] = jnp.broadcast_to(row, (out_ref.shape[0], row.shape[1]))


def kernel(sample, epoch, epoch_table):
    batch, nfeat = sample.shape
    nvocab, nemb = epoch_table.shape
    epoch_arr = jnp.asarray(epoch, jnp.int32).reshape((1,))
    nout = nfeat + nemb
    grid = (batch // _BLOCK,)
    return pl.pallas_call(
        _concat_kernel,
        grid=grid,
        in_specs=[
            pl.BlockSpec(memory_space=pltpu.SMEM),
            pl.BlockSpec((nvocab, nemb), lambda i: (0, 0)),
            pl.BlockSpec((_BLOCK, nfeat), lambda i: (i, 0)),
        ],
        out_specs=pl.BlockSpec((_BLOCK, nout), lambda i: (i, 0)),
        out_shape=jax.ShapeDtypeStruct((batch, nout), sample.dtype),
        compiler_params=pltpu.CompilerParams(
            dimension_semantics=("arbitrary",),
        ),
    )(epoch_arr, epoch_table, sample)
